# baseline (speedup 1.0000x reference)
import functools
import math

import jax
import jax.numpy as jnp
from jax.experimental import pallas as pl
from jax.experimental.pallas import tpu as pltpu

_LOG2E = 1.4426950408889634


def _attn_kernel(ptr_ref, ptrv_ref, xq_ref, xk_ref, xv_ref, b_ref,
                 wq_ref, wk_ref, wv_ref, bq_ref, bk_ref, bv_ref,
                 o_ref,
                 acc_ref, den_ref, rcnt_ref, ccnt_ref,
                 q_ref, kc_ref, vc_ref,
                 *, br, bc, nc, nptr, scale):
    i = pl.program_id(0)
    j = pl.program_id(1)
    bf = jnp.bfloat16

    @pl.when(i == 0)
    def _proj_kv():
        kc_ref[j] = (jnp.dot(xk_ref[...].astype(bf), wk_ref[...].astype(bf),
                             preferred_element_type=jnp.float32)
                     + bk_ref[...]).astype(bf)
        vc_ref[j] = (jnp.dot(xv_ref[...].astype(bf), wv_ref[...].astype(bf),
                             preferred_element_type=jnp.float32)
                     + bv_ref[...]).astype(bf)

    @pl.when(j == 0)
    def _init():
        q_ref[...] = ((jnp.dot(xq_ref[...].astype(bf),
                               wq_ref[...].astype(bf),
                               preferred_element_type=jnp.float32)
                       + bq_ref[...]) * scale).astype(bf)
        row_ids = jax.lax.broadcasted_iota(jnp.int32, (br, 1), 0) + i * br
        col_ids = (jax.lax.broadcasted_iota(jnp.int32, (nc, 1, bc), 0) * bc
                   + jax.lax.broadcasted_iota(jnp.int32, (nc, 1, bc), 2))
        rc = jnp.sum((row_ids >= ptrv_ref[...]).astype(jnp.int32),
                     axis=1, keepdims=True)
        cc = jnp.zeros((nc, 1, bc), dtype=jnp.int32)
        for t in range(nptr):
            cc += (col_ids >= ptr_ref[t]).astype(jnp.int32)
        rcnt_ref[...] = jnp.where((rc >= 1) & (rc <= nptr - 1), rc, -1)
        ccnt_ref[...] = cc
        acc_ref[...] = jnp.zeros_like(acc_ref)
        den_ref[...] = jnp.zeros_like(den_ref)

    r0 = i * br
    c0 = j * bc

    s = jax.lax.dot_general(q_ref[...], kc_ref[j],
                            (((1,), (1,)), ((), ())),
                            preferred_element_type=jnp.float32)
    s = s + b_ref[...] * _LOG2E

    rlo = clo = rhi = chi = jnp.int32(0)
    for t in range(nptr):
        p_t = ptr_ref[t]
        rlo += jnp.where(r0 >= p_t, 1, 0).astype(jnp.int32)
        rhi += jnp.where(r0 + br - 1 >= p_t, 1, 0).astype(jnp.int32)
        clo += jnp.where(c0 >= p_t, 1, 0).astype(jnp.int32)
        chi += jnp.where(c0 + bc - 1 >= p_t, 1, 0).astype(jnp.int32)
    overlap = (jnp.maximum(jnp.maximum(rlo, clo), 1)
               <= jnp.minimum(jnp.minimum(rhi, chi), nptr - 1))

    ones_col = jnp.ones((bc, 1), dtype=bf)

    @pl.when(overlap)
    def _pv():
        eq = rcnt_ref[...] == ccnt_ref[j]
        p = jnp.exp2(jnp.where(eq, s, s * -1e6)).astype(bf)
        den_ref[...] = den_ref[...] + jax.lax.dot_general(
            p, ones_col, (((1,), (0,)), ((), ())),
            preferred_element_type=jnp.float32)
        pv = jax.lax.dot_general(jnp.where(eq, p, jnp.asarray(0.0, bf)),
                                 vc_ref[j],
                                 (((1,), (0,)), ((), ())),
                                 preferred_element_type=jnp.float32)
        acc_ref[...] = acc_ref[...] + pv

    @pl.when(jnp.logical_not(overlap))
    def _masked_only():
        p = jnp.exp2(s * -1e6).astype(bf)
        den_ref[...] = den_ref[...] + jax.lax.dot_general(
            p, ones_col, (((1,), (0,)), ((), ())),
            preferred_element_type=jnp.float32)

    @pl.when(j == nc - 1)
    def _finish():
        o_ref[...] = acc_ref[...] / jnp.maximum(den_ref[...], 1e-37)


def kernel(query, key, value, edge_attr, b, edge_paths, ptr,
           Wq, bq, Wk, bk, Wv, bv, edge_vector):
    del edge_attr, edge_paths, edge_vector
    n, d_in = query.shape
    d = Wq.shape[1]
    nptr = ptr.shape[0]

    br, bc = 1024, 1024
    nr, nc = n // br, n // bc

    def _kv_idx(i, j, *_):
        return (jnp.where(i == 0, j, 0), 0)

    grid_spec = pltpu.PrefetchScalarGridSpec(
        num_scalar_prefetch=1,
        grid=(nr, nc),
        in_specs=[
            pl.BlockSpec((1, nptr), lambda i, j, *_: (0, 0)),
            pl.BlockSpec((br, d_in), lambda i, j, *_: (i, 0)),
            pl.BlockSpec((bc, d_in), _kv_idx),
            pl.BlockSpec((bc, d_in), _kv_idx),
            pl.BlockSpec((br, bc), lambda i, j, *_: (i, j)),
            pl.BlockSpec((d_in, d), lambda i, j, *_: (0, 0)),
            pl.BlockSpec((d_in, d), lambda i, j, *_: (0, 0)),
            pl.BlockSpec((d_in, d), lambda i, j, *_: (0, 0)),
            pl.BlockSpec((1, d), lambda i, j, *_: (0, 0)),
            pl.BlockSpec((1, d), lambda i, j, *_: (0, 0)),
            pl.BlockSpec((1, d), lambda i, j, *_: (0, 0)),
        ],
        out_specs=pl.BlockSpec((br, d), lambda i, j, *_: (i, 0)),
        scratch_shapes=[
            pltpu.VMEM((br, d), jnp.float32),
            pltpu.VMEM((br, 1), jnp.float32),
            pltpu.VMEM((br, 1), jnp.int32),
            pltpu.VMEM((nc, 1, bc), jnp.int32),
            pltpu.VMEM((br, d), jnp.bfloat16),
            pltpu.VMEM((nc, bc, d), jnp.bfloat16),
            pltpu.VMEM((nc, bc, d), jnp.bfloat16),
        ],
    )
    out = pl.pallas_call(
        functools.partial(_attn_kernel, br=br, bc=bc, nc=nc, nptr=nptr,
                          scale=_LOG2E / math.sqrt(d)),
        grid_spec=grid_spec,
        out_shape=jax.ShapeDtypeStruct((n, d), jnp.float32),
        compiler_params=pltpu.CompilerParams(
            dimension_semantics=("arbitrary", "arbitrary")),
    )(ptr, ptr.reshape(1, nptr), query, key, value, b, Wq, Wk, Wv,
      bq.reshape(1, d), bk.reshape(1, d), bv.reshape(1, d))
    return out


# speedup vs baseline: 1.0414x; 1.0414x over previous
import functools
import math

import jax
import jax.numpy as jnp
from jax.experimental import pallas as pl
from jax.experimental.pallas import tpu as pltpu

_LOG2E = 1.4426950408889634


def _attn_kernel(ptr_ref, ptrv_ref, xq_ref, xk_ref, xv_ref, b_ref,
                 wq_ref, wk_ref, wv_ref, bq_ref, bk_ref, bv_ref,
                 o_ref,
                 acc_ref, den_ref, rcnt_ref, ccnt_ref,
                 q_ref, kc_ref, vc_ref,
                 *, br, bc, nc, nptr, scale):
    i = pl.program_id(0)
    j = pl.program_id(1)
    bf = jnp.bfloat16

    @pl.when(i == 0)
    def _proj_kv():
        kc_ref[j] = (jnp.dot(xk_ref[...].astype(bf), wk_ref[...].astype(bf),
                             preferred_element_type=jnp.float32)
                     + bk_ref[...]).astype(bf)
        vc_ref[j] = (jnp.dot(xv_ref[...].astype(bf), wv_ref[...].astype(bf),
                             preferred_element_type=jnp.float32)
                     + bv_ref[...]).astype(bf)

    @pl.when(j == 0)
    def _init():
        q_ref[...] = ((jnp.dot(xq_ref[...].astype(bf),
                               wq_ref[...].astype(bf),
                               preferred_element_type=jnp.float32)
                       + bq_ref[...]) * scale).astype(bf)
        row_ids = jax.lax.broadcasted_iota(jnp.int32, (br, 1), 0) + i * br
        col_ids = (jax.lax.broadcasted_iota(jnp.int32, (nc, 1, bc), 0) * bc
                   + jax.lax.broadcasted_iota(jnp.int32, (nc, 1, bc), 2))
        rc = jnp.sum((row_ids >= ptrv_ref[...]).astype(jnp.int32),
                     axis=1, keepdims=True)
        cc = jnp.zeros((nc, 1, bc), dtype=jnp.int32)
        for t in range(nptr):
            cc += (col_ids >= ptr_ref[t]).astype(jnp.int32)
        rcnt_ref[...] = jnp.where((rc >= 1) & (rc <= nptr - 1), rc, -1)
        ccnt_ref[...] = cc
        acc_ref[...] = jnp.zeros_like(acc_ref)
        den_ref[...] = jnp.zeros_like(den_ref)

    r0 = i * br
    c0 = j * bc

    s = jax.lax.dot_general(q_ref[...], kc_ref[j],
                            (((1,), (1,)), ((), ())),
                            preferred_element_type=jnp.float32)
    s = s + b_ref[...] * _LOG2E

    rlo = clo = rhi = chi = jnp.int32(0)
    for t in range(nptr):
        p_t = ptr_ref[t]
        rlo += jnp.where(r0 >= p_t, 1, 0).astype(jnp.int32)
        rhi += jnp.where(r0 + br - 1 >= p_t, 1, 0).astype(jnp.int32)
        clo += jnp.where(c0 >= p_t, 1, 0).astype(jnp.int32)
        chi += jnp.where(c0 + bc - 1 >= p_t, 1, 0).astype(jnp.int32)
    overlap = (jnp.maximum(jnp.maximum(rlo, clo), 1)
               <= jnp.minimum(jnp.minimum(rhi, chi), nptr - 1))

    @pl.when(overlap)
    def _pv():
        eq = rcnt_ref[...] == ccnt_ref[j]
        p = jnp.exp2(jnp.where(eq, s, s * -1e6))
        den_ref[...] = den_ref[...] + jnp.sum(p, axis=1, keepdims=True)
        pv = jax.lax.dot_general(jnp.where(eq, p, 0.0).astype(bf), vc_ref[j],
                                 (((1,), (0,)), ((), ())),
                                 preferred_element_type=jnp.float32)
        acc_ref[...] = acc_ref[...] + pv

    @pl.when(jnp.logical_not(overlap))
    def _masked_only():
        p = jnp.exp2(s * -1e6)
        den_ref[...] = den_ref[...] + jnp.sum(p, axis=1, keepdims=True)

    @pl.when(j == nc - 1)
    def _finish():
        o_ref[...] = acc_ref[...] / jnp.maximum(den_ref[...], 1e-37)


def kernel(query, key, value, edge_attr, b, edge_paths, ptr,
           Wq, bq, Wk, bk, Wv, bv, edge_vector):
    del edge_attr, edge_paths, edge_vector
    n, d_in = query.shape
    d = Wq.shape[1]
    nptr = ptr.shape[0]

    br, bc = 2048, 512
    nr, nc = n // br, n // bc

    def _kv_idx(i, j, *_):
        return (jnp.where(i == 0, j, 0), 0)

    grid_spec = pltpu.PrefetchScalarGridSpec(
        num_scalar_prefetch=1,
        grid=(nr, nc),
        in_specs=[
            pl.BlockSpec((1, nptr), lambda i, j, *_: (0, 0)),
            pl.BlockSpec((br, d_in), lambda i, j, *_: (i, 0)),
            pl.BlockSpec((bc, d_in), _kv_idx),
            pl.BlockSpec((bc, d_in), _kv_idx),
            pl.BlockSpec((br, bc), lambda i, j, *_: (i, j)),
            pl.BlockSpec((d_in, d), lambda i, j, *_: (0, 0)),
            pl.BlockSpec((d_in, d), lambda i, j, *_: (0, 0)),
            pl.BlockSpec((d_in, d), lambda i, j, *_: (0, 0)),
            pl.BlockSpec((1, d), lambda i, j, *_: (0, 0)),
            pl.BlockSpec((1, d), lambda i, j, *_: (0, 0)),
            pl.BlockSpec((1, d), lambda i, j, *_: (0, 0)),
        ],
        out_specs=pl.BlockSpec((br, d), lambda i, j, *_: (i, 0)),
        scratch_shapes=[
            pltpu.VMEM((br, d), jnp.float32),
            pltpu.VMEM((br, 1), jnp.float32),
            pltpu.VMEM((br, 1), jnp.int32),
            pltpu.VMEM((nc, 1, bc), jnp.int32),
            pltpu.VMEM((br, d), jnp.bfloat16),
            pltpu.VMEM((nc, bc, d), jnp.bfloat16),
            pltpu.VMEM((nc, bc, d), jnp.bfloat16),
        ],
    )
    out = pl.pallas_call(
        functools.partial(_attn_kernel, br=br, bc=bc, nc=nc, nptr=nptr,
                          scale=_LOG2E / math.sqrt(d)),
        grid_spec=grid_spec,
        out_shape=jax.ShapeDtypeStruct((n, d), jnp.float32),
        compiler_params=pltpu.CompilerParams(
            dimension_semantics=("arbitrary", "arbitrary")),
    )(ptr, ptr.reshape(1, nptr), query, key, value, b, Wq, Wk, Wv,
      bq.reshape(1, d), bk.reshape(1, d), bv.reshape(1, d))
    return out
